# baseline (speedup 1.0000x reference)
import jax
import jax.numpy as jnp
from jax import lax
from jax.experimental import pallas as pl
from jax.experimental.pallas import tpu as pltpu
from jax.experimental.pallas import tpu_sc as plsc

BATCH = 1024
N_AUGS = 64
N_CLASSES = 1000
N_SUB = 8

_NC = 2
_NS = 16
_NW = _NC * _NS
_B_PER_W = BATCH // _NW
_BC = 8
_CHUNKS = _B_PER_W // _BC
_ROWS = N_SUB * _BC
_NVEC = (N_CLASSES + 15) // 16


def _body(x_hbm, idx_hbm, out_hbm, idx_v, buf, acc, sem):
    wid = lax.axis_index("s") * _NC + lax.axis_index("c")
    base = wid * _B_PER_W
    pltpu.sync_copy(idx_hbm.at[wid], idx_v)
    for chunk in range(_CHUNKS):
        row0 = base + chunk * _BC
        pltpu.async_copy(x_hbm.at[idx_v.at[chunk]], buf, sem).wait()

        for b in range(_BC):
            def _red(i, _, b=b):
                o = pl.multiple_of(i * 16, 16)
                vsum = buf[b, pl.ds(o, 16)]
                for j in range(1, N_SUB):
                    vsum = vsum + buf[j * _BC + b, pl.ds(o, 16)]
                acc[b, pl.ds(o, 16)] = vsum * 0.125
                return 0

            lax.fori_loop(0, N_CLASSES // 16, _red, 0)
            ot = N_CLASSES - 16
            vsum = buf[b, pl.ds(ot, 16)]
            for j in range(1, N_SUB):
                vsum = vsum + buf[j * _BC + b, pl.ds(ot, 16)]
            acc[b, pl.ds(ot, 16)] = vsum * 0.125
        pltpu.sync_copy(acc, out_hbm.at[pl.ds(row0, _BC)])


_sc_call = pl.kernel(
    _body,
    out_type=jax.ShapeDtypeStruct((BATCH, N_CLASSES), jnp.float32),
    mesh=plsc.VectorSubcoreMesh(core_axis_name="c", subcore_axis_name="s"),
    compiler_params=pltpu.CompilerParams(use_tc_tiling_on_sc=False),
    scratch_types=[
        pltpu.VMEM((_CHUNKS, _ROWS), jnp.int32),
        pltpu.VMEM((_ROWS, N_CLASSES), jnp.float32),
        pltpu.VMEM((_BC, N_CLASSES), jnp.float32),
        pltpu.SemaphoreType.DMA,
    ],
)


def kernel(x, idxs):
    x2 = x.reshape(BATCH * N_AUGS, N_CLASSES)
    b_abs = jnp.arange(BATCH, dtype=jnp.int32).reshape(_NW, _CHUNKS, 1, _BC)
    idx = (b_abs * N_AUGS + idxs.astype(jnp.int32)[None, None, :, None]).reshape(
        _NW, _CHUNKS, _ROWS)
    return _sc_call(x2, idx)


# speedup vs baseline: 1.3267x; 1.3267x over previous
import jax
import jax.numpy as jnp
from jax import lax
from jax.experimental import pallas as pl
from jax.experimental.pallas import tpu as pltpu
from jax.experimental.pallas import tpu_sc as plsc

BATCH = 1024
N_AUGS = 64
N_CLASSES = 1000
N_SUB = 8

_NC = 2
_NS = 16
_NW = _NC * _NS
_B_PER_W = BATCH // _NW
_NFULL = N_CLASSES // 16
_OT = N_CLASSES - 16


def _body(x_hbm, idxs_hbm, out_hbm, idxs_v, buf, acc, sem):
    wid = lax.axis_index("s") * _NC + lax.axis_index("c")
    base = wid * _B_PER_W
    pltpu.sync_copy(idxs_hbm, idxs_v.at[pl.ds(0, N_SUB)])
    ivec = idxs_v[...]
    lanes = lax.iota(jnp.int32, 16)
    a_j = [jnp.sum(jnp.where(lanes == j, ivec, 0)) for j in range(N_SUB)]
    g8 = [pl.multiple_of((a >> 3) << 3, 8) for a in a_j]
    r_j = [a & 7 for a in a_j]

    for bl in range(_B_PER_W):
        b_abs = base + bl
        descs = [
            pltpu.async_copy(x_hbm.at[b_abs, pl.ds(g8[j], 8)], buf.at[j], sem)
            for j in range(N_SUB)
        ]
        for d in descs:
            d.wait()

        br = bl % 8

        def _red(i, _, br=br):
            o = pl.multiple_of(i * 16, 16)
            vsum = buf[0, r_j[0], pl.ds(o, 16)]
            for j in range(1, N_SUB):
                vsum = vsum + buf[j, r_j[j], pl.ds(o, 16)]
            acc[br, pl.ds(o, 16)] = vsum * 0.125
            return 0

        lax.fori_loop(0, _NFULL, _red, 0)
        vsum = buf[0, r_j[0], pl.ds(_OT, 16)]
        for j in range(1, N_SUB):
            vsum = vsum + buf[j, r_j[j], pl.ds(_OT, 16)]
        acc[br, pl.ds(_OT, 16)] = vsum * 0.125

        if br == 7:
            pltpu.sync_copy(acc, out_hbm.at[pl.ds(b_abs - 7, 8)])


_sc_call = pl.kernel(
    _body,
    out_type=jax.ShapeDtypeStruct((BATCH, N_CLASSES), jnp.float32),
    mesh=plsc.VectorSubcoreMesh(core_axis_name="c", subcore_axis_name="s"),
    compiler_params=pltpu.CompilerParams(use_tc_tiling_on_sc=True,
                                         needs_layout_passes=False),
    scratch_types=[
        pltpu.VMEM((16,), jnp.int32),
        pltpu.VMEM((N_SUB, 8, N_CLASSES), jnp.float32),
        pltpu.VMEM((8, N_CLASSES), jnp.float32),
        pltpu.SemaphoreType.DMA,
    ],
)


def kernel(x, idxs):
    return _sc_call(x, idxs.astype(jnp.int32))


# speedup vs baseline: 10.8350x; 8.1667x over previous
import jax
import jax.numpy as jnp
from jax import lax
from jax.experimental import pallas as pl
from jax.experimental.pallas import tpu as pltpu
from jax.experimental.pallas import tpu_sc as plsc

BATCH = 1024
N_AUGS = 64
N_CLASSES = 1000
N_SUB = 8

_NC = 2
_NS = 16
_NW = _NC * _NS
_STRIPS = N_CLASSES // 8
_SPW = -(-_STRIPS // _NW)
_NVEC = BATCH // 16


def _body(xt_hbm, idxs_hbm, out_hbm, idxs_v, buf, obuf, sem):
    wid = lax.axis_index("s") * _NC + lax.axis_index("c")
    pltpu.sync_copy(idxs_hbm, idxs_v.at[pl.ds(0, N_SUB)])
    ivec = idxs_v[...]
    lanes = lax.iota(jnp.int32, 16)
    a_j = [jnp.sum(jnp.where(lanes == j, ivec, 0)) for j in range(N_SUB)]

    for t in range(_SPW):
        s = wid * _SPW + t

        @pl.when(s < _STRIPS)
        def _():
            row0 = pl.multiple_of(s * 8, 8)
            descs = [
                pltpu.async_copy(xt_hbm.at[a_j[j], pl.ds(row0, 8)],
                                 buf.at[j], sem)
                for j in range(N_SUB)
            ]
            for d in descs:
                d.wait()

            for r in range(8):
                def _red(i, _, r=r):
                    o = pl.multiple_of(i * 16, 16)
                    vsum = buf[0, r, pl.ds(o, 16)]
                    for j in range(1, N_SUB):
                        vsum = vsum + buf[j, r, pl.ds(o, 16)]
                    obuf[r, pl.ds(o, 16)] = vsum * 0.125
                    return 0

                lax.fori_loop(0, _NVEC, _red, 0)
            pltpu.sync_copy(obuf, out_hbm.at[pl.ds(row0, 8)])


_sc_call = pl.kernel(
    _body,
    out_type=jax.ShapeDtypeStruct((N_CLASSES, BATCH), jnp.float32),
    mesh=plsc.VectorSubcoreMesh(core_axis_name="c", subcore_axis_name="s"),
    compiler_params=pltpu.CompilerParams(use_tc_tiling_on_sc=True,
                                         needs_layout_passes=False),
    scratch_types=[
        pltpu.VMEM((16,), jnp.int32),
        pltpu.VMEM((N_SUB, 8, BATCH), jnp.float32),
        pltpu.VMEM((8, BATCH), jnp.float32),
        pltpu.SemaphoreType.DMA,
    ],
)


def kernel(x, idxs):
    xt = jnp.transpose(x, (1, 2, 0))
    out_t = _sc_call(xt, idxs.astype(jnp.int32))
    return jnp.transpose(out_t, (1, 0))


# speedup vs baseline: 11.7252x; 1.0822x over previous
import jax
import jax.numpy as jnp
from jax import lax
from jax.experimental import pallas as pl
from jax.experimental.pallas import tpu as pltpu
from jax.experimental.pallas import tpu_sc as plsc

BATCH = 1024
N_AUGS = 64
N_CLASSES = 1000
N_SUB = 8

_NC = 2
_NS = 16
_NW = _NC * _NS
_COLS = 512
_CSPLIT = BATCH // _COLS
_UNITS = (N_CLASSES // 8) * _CSPLIT
_UPW = -(-_UNITS // _NW)
_NVEC = _COLS // 16


def _body(xt_hbm, idxs_hbm, out_hbm, idxs_v, bufs, obuf, sems):
    wid = lax.axis_index("s") * _NC + lax.axis_index("c")
    pltpu.sync_copy(idxs_hbm, idxs_v.at[pl.ds(0, N_SUB)])
    ivec = idxs_v[...]
    lanes = lax.iota(jnp.int32, 16)
    a_j = [jnp.sum(jnp.where(lanes == j, ivec, 0)) for j in range(N_SUB)]

    def unit_coords(u):
        row0 = pl.multiple_of((u >> 1) * 8, 8)
        col0 = pl.multiple_of((u & 1) * _COLS, 128)
        return row0, col0

    def issue(u, slot):
        row0, col0 = unit_coords(u)
        for j in range(N_SUB):
            pltpu.async_copy(
                xt_hbm.at[a_j[j], pl.ds(row0, 8), pl.ds(col0, _COLS)],
                bufs.at[slot, j], sems.at[slot])

    def drain(slot):
        for j in range(N_SUB):
            pltpu.make_async_copy(
                xt_hbm.at[0, pl.ds(0, 8), pl.ds(0, _COLS)],
                bufs.at[slot, j], sems.at[slot]).wait()

    issue(wid, 0)
    for t in range(_UPW):
        u = wid + _NW * t
        if t + 1 < _UPW:
            nxt = wid + _NW * (t + 1)

            @pl.when(nxt < _UNITS)
            def _():
                issue(nxt, (t + 1) % 2)

        @pl.when(u < _UNITS)
        def _(t=t, u=u):
            slot = t % 2
            drain(slot)
            for r in range(8):
                def _red(i, _, r=r, slot=slot):
                    o = pl.multiple_of(i * 16, 16)
                    vsum = bufs[slot, 0, r, pl.ds(o, 16)]
                    for j in range(1, N_SUB):
                        vsum = vsum + bufs[slot, j, r, pl.ds(o, 16)]
                    obuf[r, pl.ds(o, 16)] = vsum * 0.125
                    return 0

                lax.fori_loop(0, _NVEC, _red, 0)
            row0, col0 = unit_coords(u)
            pltpu.sync_copy(obuf,
                            out_hbm.at[pl.ds(row0, 8), pl.ds(col0, _COLS)])


_sc_call = pl.kernel(
    _body,
    out_type=jax.ShapeDtypeStruct((N_CLASSES, BATCH), jnp.float32),
    mesh=plsc.VectorSubcoreMesh(core_axis_name="c", subcore_axis_name="s"),
    compiler_params=pltpu.CompilerParams(use_tc_tiling_on_sc=True,
                                         needs_layout_passes=False),
    scratch_types=[
        pltpu.VMEM((16,), jnp.int32),
        pltpu.VMEM((2, N_SUB, 8, _COLS), jnp.float32),
        pltpu.VMEM((8, _COLS), jnp.float32),
        pltpu.SemaphoreType.DMA((2,)),
    ],
)


def kernel(x, idxs):
    xt = jnp.transpose(x, (1, 2, 0))
    out_t = _sc_call(xt, idxs.astype(jnp.int32))
    return jnp.transpose(out_t, (1, 0))
